# baseline (speedup 1.0000x reference)
import functools

import jax
import jax.numpy as jnp
from jax import lax
from jax.experimental import pallas as pl
from jax.experimental.pallas import tpu as pltpu
from jax.experimental.pallas import tpu_sc as plsc

B, C, K, H, W = 8, 2, 1024, 512, 512
NUM_CELLS = 2 * B * C
LANES = 16
CHUNK = 128
NCHUNK = K // CHUNK
SUB = CHUNK // LANES

_mesh = plsc.VectorSubcoreMesh(core_axis_name="c", subcore_axis_name="s")


def _tiled_idx(x, y):
    return (((x >> 3) << 12) + ((y >> 7) << 10)
            + ((x & 7) << 7) + (y & 127))


@functools.partial(
    pl.kernel,
    out_type=jax.ShapeDtypeStruct((NUM_CELLS * LANES,), jnp.float32),
    mesh=_mesh,
    scratch_types=[
        pltpu.VMEM((4, K), jnp.int32),
        pltpu.VMEM((NCHUNK, CHUNK), jnp.int32),
        pltpu.VMEM((NCHUNK, CHUNK), jnp.int32),
        pltpu.VMEM((NCHUNK, CHUNK), jnp.float32),
        pltpu.VMEM((NCHUNK, CHUNK), jnp.float32),
        pltpu.VMEM((LANES,), jnp.float32),
        pltpu.SemaphoreType.DMA,
        pltpu.SemaphoreType.DMA,
    ],
)
def _bd_loss_sc(pred_hbm, bx0_hbm, by0_hbm, dx0_hbm, dy0_hbm,
                bx1_hbm, by1_hbm, dx1_hbm, dy1_hbm, out_hbm,
                fld_v, bidx_v, didx_v, bvals_v, dvals_v, acc_v, sem, fsem):
    cell = lax.axis_index("s") * 2 + lax.axis_index("c")
    comp = lax.div(cell, 16)
    bc = lax.rem(cell, 16)
    base = bc * (H * W)
    good_i = lax.max(1 - comp, lax.rem(cell, 2))

    fcopies0 = [
        pltpu.make_async_copy(ref.at[bc], fld_v.at[f], fsem)
        for f, ref in enumerate((bx0_hbm, by0_hbm, dx0_hbm, dy0_hbm))
    ]
    fcopies1 = [
        pltpu.make_async_copy(ref.at[bc], fld_v.at[f], fsem)
        for f, ref in enumerate((bx1_hbm, by1_hbm, dx1_hbm, dy1_hbm))
    ]

    @pl.when(comp == 0)
    def _():
        for cp in fcopies0:
            cp.start()

    @pl.when(comp == 1)
    def _():
        for cp in fcopies1:
            cp.start()

    for cp in fcopies0:
        cp.wait()

    copies = []
    for j in range(NCHUNK):
        for t in range(SUB):
            o = pl.ds(j * CHUNK + t * LANES, LANES)
            s = pl.ds(t * LANES, LANES)
            bidx_v[j, s] = base + _tiled_idx(fld_v[0, o], fld_v[1, o])
            didx_v[j, s] = base + _tiled_idx(fld_v[2, o], fld_v[3, o])
        cb = pltpu.make_async_copy(
            pred_hbm.at[bidx_v.at[j]], bvals_v.at[j], sem)
        cd = pltpu.make_async_copy(
            pred_hbm.at[didx_v.at[j]], dvals_v.at[j], sem)
        cb.start()
        cd.start()
        copies.append(cb)
        copies.append(cd)
    for cp in copies:
        cp.wait()

    lane = lax.iota(jnp.int32, LANES)
    flip = (jnp.maximum(1 - lane, 0) * good_i).astype(jnp.float32)
    acc = jnp.zeros((LANES,), jnp.float32)
    for j in range(NCHUNK):
        for t in range(SUB):
            s = pl.ds(t * LANES, LANES)
            d = bvals_v[j, s] - dvals_v[j, s]
            d2 = d * d
            if j == 0 and t == 0:
                d2 = d2 + flip * (1.0 - 2.0 * d2)
            acc = acc + d2

    acc_v[...] = acc
    pltpu.sync_copy(acc_v, out_hbm.at[pl.ds(cell * LANES, LANES)])


def kernel(prediction, intervals_comp_0, intervals_comp_1):
    pred_t = prediction.reshape(B, C, H // 8, 8, W // 128, 128)
    pred_t = pred_t.transpose(0, 1, 2, 4, 3, 5).reshape(-1)

    def fields(ints):
        return [ints[:, :, :, p, q].reshape(B * C, K)
                for p in (0, 1) for q in (0, 1)]

    partials = _bd_loss_sc(
        pred_t, *fields(intervals_comp_0), *fields(intervals_comp_1))
    return jnp.sum(partials)


# speedup vs baseline: 1.1406x; 1.1406x over previous
import functools

import jax
import jax.numpy as jnp
from jax import lax
from jax.experimental import pallas as pl
from jax.experimental.pallas import tpu as pltpu
from jax.experimental.pallas import tpu_sc as plsc

B, C, K, H, W = 8, 2, 1024, 512, 512
NUM_CELLS = 2 * B * C
LANES = 16
CHUNK = 128
NCHUNK = K // CHUNK
SUB = CHUNK // LANES

_mesh = plsc.VectorSubcoreMesh(core_axis_name="c", subcore_axis_name="s")


def _tiled_idx(x, y):
    return (((x >> 3) << 12) + ((y >> 7) << 10)
            + ((x & 7) << 7) + (y & 127))


@functools.partial(
    pl.kernel,
    out_type=jax.ShapeDtypeStruct((NUM_CELLS * LANES,), jnp.float32),
    mesh=_mesh,
    scratch_types=[
        pltpu.VMEM((4, K), jnp.int32),
        pltpu.VMEM((NCHUNK, CHUNK), jnp.int32),
        pltpu.VMEM((NCHUNK, CHUNK), jnp.int32),
        pltpu.VMEM((NCHUNK, CHUNK), jnp.float32),
        pltpu.VMEM((NCHUNK, CHUNK), jnp.float32),
        pltpu.VMEM((LANES,), jnp.float32),
        pltpu.SemaphoreType.DMA,
        pltpu.SemaphoreType.DMA,
    ],
)
def _bd_loss_sc(pred_hbm, fld_hbm, out_hbm,
                fld_v, bidx_v, didx_v, bvals_v, dvals_v, acc_v, sem, fsem):
    cell = lax.axis_index("s") * 2 + lax.axis_index("c")
    comp = lax.div(cell, 16)
    bc = lax.rem(cell, 16)
    base = bc * (H * W)
    good_i = lax.max(1 - comp, lax.rem(cell, 2))

    fcopies = [
        pltpu.make_async_copy(
            fld_hbm.at[f * NUM_CELLS + cell], fld_v.at[f], fsem)
        for f in range(4)
    ]
    for cp in fcopies:
        cp.start()
    for cp in fcopies:
        cp.wait()

    copies = []
    for j in range(NCHUNK):
        for t in range(SUB):
            o = pl.ds(j * CHUNK + t * LANES, LANES)
            s = pl.ds(t * LANES, LANES)
            bidx_v[j, s] = base + _tiled_idx(fld_v[0, o], fld_v[1, o])
            didx_v[j, s] = base + _tiled_idx(fld_v[2, o], fld_v[3, o])
        cb = pltpu.make_async_copy(
            pred_hbm.at[bidx_v.at[j]], bvals_v.at[j], sem)
        cd = pltpu.make_async_copy(
            pred_hbm.at[didx_v.at[j]], dvals_v.at[j], sem)
        cb.start()
        cd.start()
        copies.append(cb)
        copies.append(cd)
    for cp in copies:
        cp.wait()

    lane = lax.iota(jnp.int32, LANES)
    flip = (jnp.maximum(1 - lane, 0) * good_i).astype(jnp.float32)
    acc = jnp.zeros((LANES,), jnp.float32)
    for j in range(NCHUNK):
        for t in range(SUB):
            s = pl.ds(t * LANES, LANES)
            d = bvals_v[j, s] - dvals_v[j, s]
            d2 = d * d
            if j == 0 and t == 0:
                d2 = d2 + flip * (1.0 - 2.0 * d2)
            acc = acc + d2

    acc_v[...] = acc
    pltpu.sync_copy(acc_v, out_hbm.at[pl.ds(cell * LANES, LANES)])


def kernel(prediction, intervals_comp_0, intervals_comp_1):
    pred_t = prediction.reshape(B, C, H // 8, 8, W // 128, 128)
    pred_t = pred_t.transpose(0, 1, 2, 4, 3, 5).reshape(-1)

    def field(p, q):
        return jnp.concatenate([
            intervals_comp_0[:, :, :, p, q].reshape(B * C, K),
            intervals_comp_1[:, :, :, p, q].reshape(B * C, K),
        ])

    fld = jnp.concatenate(
        [field(0, 0), field(0, 1), field(1, 0), field(1, 1)])
    partials = _bd_loss_sc(pred_t, fld)
    return jnp.sum(partials)


# speedup vs baseline: 1.1491x; 1.0075x over previous
import functools

import jax
import jax.numpy as jnp
from jax import lax
from jax.experimental import pallas as pl
from jax.experimental.pallas import tpu as pltpu
from jax.experimental.pallas import tpu_sc as plsc

B, C, K, H, W = 8, 2, 1024, 512, 512
NUM_CELLS = 2 * B * C
LANES = 16
CHUNK = 128
NCHUNK = K // CHUNK
SUB = CHUNK // LANES

_mesh = plsc.VectorSubcoreMesh(core_axis_name="c", subcore_axis_name="s")


def _tiled_idx(x, y):
    return (((x >> 3) << 12) + ((y >> 7) << 10)
            + ((x & 7) << 7) + (y & 127))


@functools.partial(
    pl.kernel,
    out_type=jax.ShapeDtypeStruct((NUM_CELLS * LANES,), jnp.float32),
    mesh=_mesh,
    scratch_types=[
        pltpu.VMEM((4 * K,), jnp.int32),
        pltpu.VMEM((NCHUNK, CHUNK), jnp.int32),
        pltpu.VMEM((NCHUNK, CHUNK), jnp.int32),
        pltpu.VMEM((NCHUNK, CHUNK), jnp.float32),
        pltpu.VMEM((NCHUNK, CHUNK), jnp.float32),
        pltpu.VMEM((LANES,), jnp.float32),
        pltpu.SemaphoreType.DMA,
        pltpu.SemaphoreType.DMA,
    ],
)
def _bd_loss_sc(pred_hbm, fld_hbm, out_hbm,
                fld_v, bidx_v, didx_v, bvals_v, dvals_v, acc_v, sem, fsem):
    cell = lax.axis_index("s") * 2 + lax.axis_index("c")
    comp = lax.div(cell, 16)
    bc = lax.rem(cell, 16)
    base = bc * (H * W)
    good_i = lax.max(1 - comp, lax.rem(cell, 2))

    cpf = pltpu.make_async_copy(fld_hbm.at[cell], fld_v, fsem)
    cpf.start()
    cpf.wait()

    copies = []
    for j in range(NCHUNK):
        for t in range(SUB):
            o = j * CHUNK + t * LANES
            s = pl.ds(t * LANES, LANES)
            bidx_v[j, s] = base + _tiled_idx(
                fld_v[pl.ds(o, LANES)], fld_v[pl.ds(K + o, LANES)])
            didx_v[j, s] = base + _tiled_idx(
                fld_v[pl.ds(2 * K + o, LANES)], fld_v[pl.ds(3 * K + o, LANES)])
        cb = pltpu.make_async_copy(
            pred_hbm.at[bidx_v.at[j]], bvals_v.at[j], sem)
        cd = pltpu.make_async_copy(
            pred_hbm.at[didx_v.at[j]], dvals_v.at[j], sem)
        cb.start()
        cd.start()
        copies.append(cb)
        copies.append(cd)
    for cp in copies:
        cp.wait()

    lane = lax.iota(jnp.int32, LANES)
    flip = (jnp.maximum(1 - lane, 0) * good_i).astype(jnp.float32)
    acc = jnp.zeros((LANES,), jnp.float32)
    for j in range(NCHUNK):
        for t in range(SUB):
            s = pl.ds(t * LANES, LANES)
            d = bvals_v[j, s] - dvals_v[j, s]
            d2 = d * d
            if j == 0 and t == 0:
                d2 = d2 + flip * (1.0 - 2.0 * d2)
            acc = acc + d2

    acc_v[...] = acc
    pltpu.sync_copy(acc_v, out_hbm.at[pl.ds(cell * LANES, LANES)])


def kernel(prediction, intervals_comp_0, intervals_comp_1):
    pred_t = prediction.reshape(B, C, H // 8, 8, W // 128, 128)
    pred_t = pred_t.transpose(0, 1, 2, 4, 3, 5).reshape(-1)

    def field(p, q):
        return jnp.concatenate([
            intervals_comp_0[:, :, :, p, q].reshape(B * C, K),
            intervals_comp_1[:, :, :, p, q].reshape(B * C, K),
        ])

    fld = jnp.concatenate(
        [field(0, 0), field(0, 1), field(1, 0), field(1, 1)], axis=1)
    partials = _bd_loss_sc(pred_t, fld)
    return jnp.sum(partials)
